# Initial kernel scaffold; baseline (speedup 1.0000x reference)
import functools

import jax
import jax.numpy as jnp
from jax import lax
from jax.experimental import pallas as pl
from jax.experimental.pallas import tpu as pltpu
from jax.experimental.pallas import tpu_sc as plsc

N_NODES = 10000
N_EDGES = 320000
D_IN = 128
D_HID = 16
D_OUT = 64

NC = 2
NS = 16
NW = NC * NS
CHUNK = 128
NCHUNK = -(-N_EDGES // (NW * CHUNK))
E_PAD = NW * NCHUNK * CHUNK

ROWS_PER_SUB = N_NODES // NS
ZB_ROWS = 125


def _sc_edge_pass_body(feat, srcr, dstr, ewr, out, acc, src_v, dst_v, ew_v,
                       rows, zb, sem):
    c = lax.axis_index("c")
    s = lax.axis_index("s")

    zrow = jnp.zeros((16,), jnp.float32)
    for i in range(ZB_ROWS):
        zb[i, :] = zrow
    for k in range(ROWS_PER_SUB // ZB_ROWS):
        pltpu.sync_copy(zb, acc.at[pl.ds(ROWS_PER_SUB * s + ZB_ROWS * k,
                                         ZB_ROWS)])
    plsc.subcore_barrier()

    w = s * NC + c
    lanes = lax.iota(jnp.int32, 16)

    def chunk(j, carry):
        pltpu.sync_copy(srcr.at[w, j], src_v)
        pltpu.sync_copy(dstr.at[w, j], dst_v)
        pltpu.sync_copy(ewr.at[w, j], ew_v)
        pltpu.async_copy(feat.at[src_v], rows, sem).wait()
        for g in range(CHUNK // 16):
            ew_vec = ew_v[pl.ds(16 * g, 16)]
            ridx = 16 * g + lanes
            for f in range(D_HID):
                cidx = jnp.full((16,), f, jnp.int32)
                col = plsc.load_gather(rows, [ridx, cidx])
                plsc.store_scatter(rows, [ridx, cidx], col * ew_vec)
        pltpu.sync_copy(rows, acc.at[dst_v], add=True)
        return carry

    lax.fori_loop(0, NCHUNK, chunk, 0)
    plsc.subcore_barrier()

    for k in range(ROWS_PER_SUB // ZB_ROWS):
        base = ROWS_PER_SUB * s + ZB_ROWS * k
        pltpu.sync_copy(acc.at[pl.ds(base, ZB_ROWS)], zb)
        pltpu.sync_copy(zb, out.at[c, pl.ds(base, ZB_ROWS)])


_sc_edge_pass = pl.kernel(
    _sc_edge_pass_body,
    out_type=jax.ShapeDtypeStruct((NC, N_NODES, D_HID), jnp.float32),
    mesh=plsc.VectorSubcoreMesh(core_axis_name="c", subcore_axis_name="s"),
    scratch_types=[
        pltpu.VMEM_SHARED((N_NODES, D_HID), jnp.float32),
        pltpu.VMEM((CHUNK,), jnp.int32),
        pltpu.VMEM((CHUNK,), jnp.int32),
        pltpu.VMEM((CHUNK,), jnp.float32),
        pltpu.VMEM((CHUNK, D_HID), jnp.float32),
        pltpu.VMEM((ZB_ROWS, D_HID), jnp.float32),
        pltpu.SemaphoreType.DMA,
    ],
)


def _mm1_body(x_ref, w_ref, o_ref):
    o_ref[...] = jnp.dot(x_ref[...], w_ref[...],
                         preferred_element_type=jnp.float32)


def _tc_mm1(x, W1):
    blk = 1000
    return pl.pallas_call(
        _mm1_body,
        grid=(N_NODES // blk,),
        in_specs=[
            pl.BlockSpec((blk, D_IN), lambda i: (i, 0)),
            pl.BlockSpec((D_IN, D_HID), lambda i: (0, 0)),
        ],
        out_specs=pl.BlockSpec((blk, D_HID), lambda i: (i, 0)),
        out_shape=jax.ShapeDtypeStruct((N_NODES, D_HID), jnp.float32),
    )(x, W1)


def _relu_body(p_ref, b_ref, o_ref):
    o_ref[...] = jnp.maximum(p_ref[0] + p_ref[1] + b_ref[...], 0.0)


def _tc_combine_relu(parts, b1):
    blk = 1000
    return pl.pallas_call(
        _relu_body,
        grid=(N_NODES // blk,),
        in_specs=[
            pl.BlockSpec((NC, blk, D_HID), lambda i: (0, i, 0)),
            pl.BlockSpec((1, D_HID), lambda i: (0, 0)),
        ],
        out_specs=pl.BlockSpec((blk, D_HID), lambda i: (i, 0)),
        out_shape=jax.ShapeDtypeStruct((N_NODES, D_HID), jnp.float32),
    )(parts, b1.reshape(1, D_HID))


def _mm2_body(p_ref, w_ref, b_ref, o_ref):
    a = p_ref[0] + p_ref[1]
    o_ref[...] = jnp.dot(a, w_ref[...],
                         preferred_element_type=jnp.float32) + b_ref[...]


def _tc_combine_mm2(parts, W2, b2):
    blk = 1000
    return pl.pallas_call(
        _mm2_body,
        grid=(N_NODES // blk,),
        in_specs=[
            pl.BlockSpec((NC, blk, D_HID), lambda i: (0, i, 0)),
            pl.BlockSpec((D_HID, D_OUT), lambda i: (0, 0)),
            pl.BlockSpec((1, D_OUT), lambda i: (0, 0)),
        ],
        out_specs=pl.BlockSpec((blk, D_OUT), lambda i: (i, 0)),
        out_shape=jax.ShapeDtypeStruct((N_NODES, D_OUT), jnp.float32),
    )(parts, W2, b2.reshape(1, D_OUT))


def kernel(x, edge_index, edge_weight, W1, b1, W2, b2):
    pad = E_PAD - N_EDGES
    src = jnp.concatenate(
        [edge_index[0].astype(jnp.int32), jnp.zeros((pad,), jnp.int32)])
    dst = jnp.concatenate(
        [edge_index[1].astype(jnp.int32), jnp.zeros((pad,), jnp.int32)])
    ew = jnp.concatenate(
        [edge_weight.astype(jnp.float32), jnp.zeros((pad,), jnp.float32)])
    srcr = src.reshape(NW, NCHUNK, CHUNK)
    dstr = dst.reshape(NW, NCHUNK, CHUNK)
    ewr = ew.reshape(NW, NCHUNK, CHUNK)

    h1 = _tc_mm1(x, W1)
    a1 = _sc_edge_pass(h1, srcr, dstr, ewr)
    h = _tc_combine_relu(a1, b1)
    a2 = _sc_edge_pass(h, srcr, dstr, ewr)
    return _tc_combine_mm2(a2, W2, b2)


# speedup vs baseline: 6.6865x; 6.6865x over previous
import functools

import jax
import jax.numpy as jnp
from jax import lax
from jax.experimental import pallas as pl
from jax.experimental.pallas import tpu as pltpu
from jax.experimental.pallas import tpu_sc as plsc

N_NODES = 10000
N_EDGES = 320000
D_IN = 128
D_HID = 16
D_OUT = 64

NC = 2
NS = 16
NW = NC * NS
CHUNK = 128
NCHUNK = -(-N_EDGES // (NW * CHUNK))
E_PAD = NW * NCHUNK * CHUNK

N_PAD = 10240
ROWS_PER_SUB = N_PAD // NS
ZB_ROWS = 128


def _sc_edge_pass_body(feat, srcr, dstr, ewr, out, acc, src_v, dst_v, ew_v,
                       rows, zb, sem):
    c = lax.axis_index("c")
    s = lax.axis_index("s")

    zrow = jnp.zeros((16,), jnp.float32)
    for i in range(ZB_ROWS):
        zb[i, :] = zrow
    for k in range(ROWS_PER_SUB // ZB_ROWS):
        pltpu.sync_copy(zb, acc.at[pl.ds(ROWS_PER_SUB * s + ZB_ROWS * k,
                                         ZB_ROWS)])
    plsc.subcore_barrier()

    w = s * NC + c
    lanes = lax.iota(jnp.int32, 16)

    def chunk(j, carry):
        pltpu.sync_copy(srcr.at[w, j], src_v)
        pltpu.sync_copy(dstr.at[w, j], dst_v)
        pltpu.sync_copy(ewr.at[w, j], ew_v)
        pltpu.async_copy(feat.at[src_v], rows, sem).wait()
        for g in range(CHUNK // 16):
            ew_vec = ew_v[pl.ds(16 * g, 16)]
            for l in range(16):
                e = 16 * g + l
                rows[e, :] = rows[e, :] * jnp.full((16,), ew_vec[l],
                                                   jnp.float32)
        pltpu.sync_copy(rows, acc.at[dst_v], add=True)
        return carry

    lax.fori_loop(0, NCHUNK, chunk, 0)
    plsc.subcore_barrier()

    for k in range(ROWS_PER_SUB // ZB_ROWS):
        base = ROWS_PER_SUB * s + ZB_ROWS * k
        pltpu.sync_copy(acc.at[pl.ds(base, ZB_ROWS)], zb)
        pltpu.sync_copy(zb, out.at[c, pl.ds(base, ZB_ROWS)])


@functools.cache
def _sc_edge_pass():
  return pl.kernel(
    _sc_edge_pass_body,
    out_type=jax.ShapeDtypeStruct((NC, N_PAD, D_HID), jnp.float32),
    mesh=plsc.VectorSubcoreMesh(core_axis_name="c", subcore_axis_name="s",
                                num_cores=NC, num_subcores=NS),
    scratch_types=[
        pltpu.VMEM_SHARED((N_PAD, D_HID), jnp.float32),
        pltpu.VMEM((CHUNK,), jnp.int32),
        pltpu.VMEM((CHUNK,), jnp.int32),
        pltpu.VMEM((CHUNK,), jnp.float32),
        pltpu.VMEM((CHUNK, D_HID), jnp.float32),
        pltpu.VMEM((ZB_ROWS, D_HID), jnp.float32),
        pltpu.SemaphoreType.DMA,
    ],
    compiler_params=pltpu.CompilerParams(use_tc_tiling_on_sc=False),
  )


def _mm1_body(x_ref, w_ref, o_ref):
    o_ref[...] = jnp.dot(x_ref[...], w_ref[...],
                         preferred_element_type=jnp.float32)


def _tc_mm1(x, W1):
    blk = 1000
    return pl.pallas_call(
        _mm1_body,
        grid=(N_NODES // blk,),
        in_specs=[
            pl.BlockSpec((blk, D_IN), lambda i: (i, 0)),
            pl.BlockSpec((D_IN, D_HID), lambda i: (0, 0)),
        ],
        out_specs=pl.BlockSpec((blk, D_HID), lambda i: (i, 0)),
        out_shape=jax.ShapeDtypeStruct((N_NODES, D_HID), jnp.float32),
    )(x, W1)


def _relu_body(p_ref, b_ref, o_ref):
    o_ref[...] = jnp.maximum(p_ref[0] + p_ref[1] + b_ref[...], 0.0)


def _tc_combine_relu(parts, b1):
    blk = 1000
    return pl.pallas_call(
        _relu_body,
        grid=(N_NODES // blk,),
        in_specs=[
            pl.BlockSpec((NC, blk, D_HID), lambda i: (0, i, 0)),
            pl.BlockSpec((1, D_HID), lambda i: (0, 0)),
        ],
        out_specs=pl.BlockSpec((blk, D_HID), lambda i: (i, 0)),
        out_shape=jax.ShapeDtypeStruct((N_NODES, D_HID), jnp.float32),
    )(parts, b1.reshape(1, D_HID))


def _mm2_body(p_ref, w_ref, b_ref, o_ref):
    a = p_ref[0] + p_ref[1]
    o_ref[...] = jnp.dot(a, w_ref[...],
                         preferred_element_type=jnp.float32) + b_ref[...]


def _tc_combine_mm2(parts, W2, b2):
    blk = 1000
    return pl.pallas_call(
        _mm2_body,
        grid=(N_NODES // blk,),
        in_specs=[
            pl.BlockSpec((NC, blk, D_HID), lambda i: (0, i, 0)),
            pl.BlockSpec((D_HID, D_OUT), lambda i: (0, 0)),
            pl.BlockSpec((1, D_OUT), lambda i: (0, 0)),
        ],
        out_specs=pl.BlockSpec((blk, D_OUT), lambda i: (i, 0)),
        out_shape=jax.ShapeDtypeStruct((N_NODES, D_OUT), jnp.float32),
    )(parts, W2, b2.reshape(1, D_OUT))


def kernel(x, edge_index, edge_weight, W1, b1, W2, b2):
    pad = E_PAD - N_EDGES
    src = jnp.concatenate(
        [edge_index[0].astype(jnp.int32), jnp.zeros((pad,), jnp.int32)])
    dst = jnp.concatenate(
        [edge_index[1].astype(jnp.int32), jnp.zeros((pad,), jnp.int32)])
    ew = jnp.concatenate(
        [edge_weight.astype(jnp.float32), jnp.zeros((pad,), jnp.float32)])
    srcr = src.reshape(NW, NCHUNK, CHUNK)
    dstr = dst.reshape(NW, NCHUNK, CHUNK)
    ewr = ew.reshape(NW, NCHUNK, CHUNK)

    sc_pass = _sc_edge_pass()
    h1 = _tc_mm1(x, W1)
    a1 = sc_pass(h1, srcr, dstr, ewr)
    h = _tc_combine_relu(a1, b1)
    a2 = sc_pass(h, srcr, dstr, ewr)
    return _tc_combine_mm2(a2, W2, b2)
